# baseline (device time: 20295 ns/iter reference)
import jax
import jax.numpy as jnp
from jax import lax
from jax.experimental import pallas as pl
from jax.experimental.pallas import tpu as pltpu

N_DEV = 4
N_TOK = 1024
D_MODEL = 256
D_HID = 512
N_EXP = 16
E_LOCAL = N_EXP // N_DEV
M_PER = N_TOK // N_DEV


def kernel(x, router_W, route_idx, expert_W):
    def body(x_ref, rw_ref, idx_ref, ew_ref, out_ref,
             w_ref, send_ref, recv_ref, send_sems, recv_sems):
        my_pos = lax.axis_index("i")

        barrier_sem = pltpu.get_barrier_semaphore()
        for r in range(1, N_DEV):
            pl.semaphore_signal(
                barrier_sem, inc=1,
                device_id=(lax.rem(my_pos + r, N_DEV),),
                device_id_type=pl.DeviceIdType.MESH,
            )
        pl.semaphore_wait(barrier_sem, N_DEV - 1)

        scores = jnp.dot(x_ref[:, :], rw_ref[:, :],
                         preferred_element_type=jnp.float32,
                         precision=lax.Precision.HIGHEST)
        s_max = jnp.max(scores, axis=-1, keepdims=True)
        e = jnp.exp(scores - s_max)
        probs = e / jnp.sum(e, axis=-1, keepdims=True)

        idx = idx_ref[:, :]
        e_iota = lax.broadcasted_iota(jnp.int32, (N_TOK, N_EXP), 1)
        hit0 = idx[:, 0:1] == e_iota
        hit1 = idx[:, 1:2] == e_iota
        g0 = jnp.sum(jnp.where(hit0, probs, 0.0), axis=-1, keepdims=True)
        g1 = jnp.sum(jnp.where(hit1, probs, 0.0), axis=-1, keepdims=True)
        w_ref[:, :] = jnp.where(hit0 | hit1, probs, 0.0) / (g0 + g1)

        eww = ew_ref[:, :, :].astype(jnp.bfloat16).reshape(
            E_LOCAL * D_MODEL, D_HID)

        def tile_partial(rs, m):
            xb = x_ref[pl.ds(rs, m), :]
            wb = w_ref[pl.ds(rs, m), :]
            t_iota = lax.broadcasted_iota(jnp.int32, (m, N_EXP), 1)
            parts = []
            for j in range(E_LOCAL):
                ge = my_pos * E_LOCAL + j
                coeff = jnp.sum(jnp.where(t_iota == ge, wb, 0.0),
                                axis=-1, keepdims=True)
                parts.append((xb * coeff).astype(jnp.bfloat16))
            return jnp.dot(jnp.concatenate(parts, axis=1), eww,
                           preferred_element_type=jnp.float32)

        H_PER = M_PER // 2
        rdmas = []
        for r in range(1, N_DEV):
            dst = lax.rem(my_pos + r, N_DEV)
            for h in range(2):
                hs = h * H_PER
                send_ref[r - 1, pl.ds(hs, H_PER), :] = (
                    tile_partial(dst * M_PER + hs, H_PER)
                    .astype(jnp.bfloat16))
                rdma = pltpu.make_async_remote_copy(
                    src_ref=send_ref.at[r - 1, pl.ds(hs, H_PER), :],
                    dst_ref=recv_ref.at[r - 1, pl.ds(hs, H_PER), :],
                    send_sem=send_sems.at[r - 1, h],
                    recv_sem=recv_sems.at[r - 1, h],
                    device_id=(dst,),
                    device_id_type=pl.DeviceIdType.MESH,
                )
                rdma.start()
                rdmas.append(rdma)

        total = tile_partial(my_pos * M_PER, M_PER)
        for rdma in rdmas:
            rdma.wait_recv()
        for r in range(1, N_DEV):
            total = total + recv_ref[r - 1, :, :].astype(jnp.float32)
        out_ref[:, :] = total

        for rdma in rdmas:
            rdma.wait_send()

    return pl.pallas_call(
        body,
        out_shape=jax.ShapeDtypeStruct((M_PER, D_HID), jnp.float32),
        in_specs=[
            pl.BlockSpec(memory_space=pltpu.VMEM),
            pl.BlockSpec(memory_space=pltpu.VMEM),
            pl.BlockSpec(memory_space=pltpu.VMEM),
            pl.BlockSpec(memory_space=pltpu.VMEM),
        ],
        out_specs=pl.BlockSpec(memory_space=pltpu.VMEM),
        scratch_shapes=[
            pltpu.VMEM((N_TOK, N_EXP), jnp.float32),
            pltpu.VMEM((N_DEV - 1, M_PER, D_HID), jnp.bfloat16),
            pltpu.VMEM((N_DEV - 1, M_PER, D_HID), jnp.bfloat16),
            pltpu.SemaphoreType.DMA((N_DEV - 1, 2)),
            pltpu.SemaphoreType.DMA((N_DEV - 1, 2)),
        ],
        compiler_params=pltpu.CompilerParams(collective_id=0),
    )(x, router_W, route_idx, expert_W)


# device time: 18958 ns/iter; 1.0705x vs baseline; 1.0705x over previous
import jax
import jax.numpy as jnp
from jax import lax
from jax.experimental import pallas as pl
from jax.experimental.pallas import tpu as pltpu

N_DEV = 4
N_TOK = 1024
D_MODEL = 256
D_HID = 512
N_EXP = 16
E_LOCAL = N_EXP // N_DEV
M_PER = N_TOK // N_DEV


def kernel(x, router_W, route_idx, expert_W):
    def body(x_ref, rw_ref, idx_ref, ew_ref, out_ref,
             send_ref, recv_ref, send_sems, recv_sems):
        my_pos = lax.axis_index("i")

        barrier_sem = pltpu.get_barrier_semaphore()
        for r in range(1, N_DEV):
            pl.semaphore_signal(
                barrier_sem, inc=1,
                device_id=(lax.rem(my_pos + r, N_DEV),),
                device_id_type=pl.DeviceIdType.MESH,
            )
        pl.semaphore_wait(barrier_sem, N_DEV - 1)

        eww = ew_ref[:, :, :].astype(jnp.bfloat16).reshape(
            E_LOCAL * D_MODEL, D_HID)

        def tile_partial(rs, m):
            xb = x_ref[pl.ds(rs, m), :]
            scores = jnp.dot(xb, rw_ref[:, :],
                             preferred_element_type=jnp.float32,
                             precision=lax.Precision.HIGHEST)
            s_max = jnp.max(scores, axis=-1, keepdims=True)
            e = jnp.exp(scores - s_max)
            probs = e / jnp.sum(e, axis=-1, keepdims=True)
            idx = idx_ref[pl.ds(rs, m), :]
            t_iota = lax.broadcasted_iota(jnp.int32, (m, N_EXP), 1)
            hit0 = idx[:, 0:1] == t_iota
            hit1 = idx[:, 1:2] == t_iota
            g0 = jnp.sum(jnp.where(hit0, probs, 0.0), axis=-1, keepdims=True)
            g1 = jnp.sum(jnp.where(hit1, probs, 0.0), axis=-1, keepdims=True)
            wb = jnp.where(hit0 | hit1, probs, 0.0) / (g0 + g1)
            parts = []
            for j in range(E_LOCAL):
                ge = my_pos * E_LOCAL + j
                coeff = jnp.sum(jnp.where(t_iota == ge, wb, 0.0),
                                axis=-1, keepdims=True)
                parts.append((xb * coeff).astype(jnp.bfloat16))
            return jnp.dot(jnp.concatenate(parts, axis=1), eww,
                           preferred_element_type=jnp.float32)

        H_PER = M_PER // 2
        rdmas = []
        for r in range(1, N_DEV):
            dst = lax.rem(my_pos + r, N_DEV)
            for h in range(2):
                hs = h * H_PER
                send_ref[r - 1, pl.ds(hs, H_PER), :] = (
                    tile_partial(dst * M_PER + hs, H_PER)
                    .astype(jnp.bfloat16))
                rdma = pltpu.make_async_remote_copy(
                    src_ref=send_ref.at[r - 1, pl.ds(hs, H_PER), :],
                    dst_ref=recv_ref.at[r - 1, pl.ds(hs, H_PER), :],
                    send_sem=send_sems.at[r - 1, h],
                    recv_sem=recv_sems.at[r - 1, h],
                    device_id=(dst,),
                    device_id_type=pl.DeviceIdType.MESH,
                )
                rdma.start()
                rdmas.append(rdma)

        acc = [tile_partial(my_pos * M_PER + h * H_PER, H_PER)
               for h in range(2)]
        for r in range(1, N_DEV):
            for h in range(2):
                rdmas[(r - 1) * 2 + h].wait_recv()
                acc[h] = acc[h] + recv_ref[
                    r - 1, pl.ds(h * H_PER, H_PER), :].astype(jnp.float32)
                if r == N_DEV - 1:
                    out_ref[pl.ds(h * H_PER, H_PER), :] = acc[h]

        for rdma in rdmas:
            rdma.wait_send()

    return pl.pallas_call(
        body,
        out_shape=jax.ShapeDtypeStruct((M_PER, D_HID), jnp.float32),
        in_specs=[
            pl.BlockSpec(memory_space=pltpu.VMEM),
            pl.BlockSpec(memory_space=pltpu.VMEM),
            pl.BlockSpec(memory_space=pltpu.VMEM),
            pl.BlockSpec(memory_space=pltpu.VMEM),
        ],
        out_specs=pl.BlockSpec(memory_space=pltpu.VMEM),
        scratch_shapes=[
            pltpu.VMEM((N_DEV - 1, M_PER, D_HID), jnp.bfloat16),
            pltpu.VMEM((N_DEV - 1, M_PER, D_HID), jnp.bfloat16),
            pltpu.SemaphoreType.DMA((N_DEV - 1, 2)),
            pltpu.SemaphoreType.DMA((N_DEV - 1, 2)),
        ],
        compiler_params=pltpu.CompilerParams(collective_id=0),
    )(x, router_W, route_idx, expert_W)
